# baseline (speedup 1.0000x reference)
import jax
import jax.numpy as jnp
from jax import lax
from jax.experimental import pallas as pl
from jax.experimental.pallas import tpu as pltpu
from jax.experimental.pallas import tpu_sc as plsc

N = 10000
D = 128
E = 320000
NPAD = 10240
NC = 2
NS = 16
NW = NC * NS
CHUNK = 128
CPT = 80
EPAD = NW * CPT * CHUNK
RPT = NPAD // NS

_MESH = plsc.VectorSubcoreMesh(core_axis_name="c", subcore_axis_name="s")


def _make_segsum(with_deg):
    def body(y_hbm, edges_hbm, out_hbm, *rest):
        if with_deg:
            (deg_hbm, src_v, dst_v, rows_v, zb, ones_v, dwb, acc_sh,
             deg_sh, sem_a) = rest
        else:
            (src_v, dst_v, rows_v, zb, acc_sh, sem_a) = rest
        c = lax.axis_index("c")
        s = lax.axis_index("s")
        wid = c * NS + s
        row0 = s * RPT

        for i in range(16):
            for j in range(D // 16):
                zb[i, pl.ds(j * 16, 16)] = jnp.zeros((16,), jnp.float32)
        if with_deg:
            for j in range(CHUNK // 16):
                ones_v[pl.ds(j * 16, 16)] = jnp.ones((16,), jnp.float32)
            for j in range(RPT // 16):
                dwb[pl.ds(j * 16, 16)] = jnp.zeros((16,), jnp.float32)

        def zloop(i, carry):
            pltpu.sync_copy(zb, acc_sh.at[pl.ds(row0 + i * 16, 16)])
            return carry
        lax.fori_loop(0, RPT // 16, zloop, 0)
        if with_deg:
            pltpu.sync_copy(dwb, deg_sh.at[pl.ds(row0, RPT)])

        pltpu.sync_copy(edges_hbm.at[wid], dst_v)

        def uloop(i, carry):
            def inner(j, carry2):
                e = dst_v[i, pl.ds(j * 16, 16)]
                src_v[i, pl.ds(j * 16, 16)] = jax.lax.shift_right_logical(
                    e, jnp.full((16,), 14, jnp.int32))
                dst_v[i, pl.ds(j * 16, 16)] = jax.lax.bitwise_and(
                    e, jnp.full((16,), 16383, jnp.int32))
                return carry2
            return lax.fori_loop(0, CHUNK // 16, inner, carry)
        lax.fori_loop(0, CPT, uloop, 0)
        plsc.subcore_barrier()

        def eloop(j, carry):
            pltpu.async_copy(y_hbm.at[src_v.at[j]], rows_v, sem_a).wait()
            pltpu.sync_copy(rows_v, acc_sh.at[dst_v.at[j]], add=True)
            if with_deg:
                pltpu.sync_copy(ones_v, deg_sh.at[dst_v.at[j]], add=True)
            return carry
        lax.fori_loop(0, CPT, eloop, 0)

        plsc.subcore_barrier()

        out0 = c * NPAD + row0

        def wloop(i, carry):
            r = i * CHUNK
            pltpu.sync_copy(acc_sh.at[pl.ds(row0 + r, CHUNK)], rows_v)
            pltpu.sync_copy(rows_v, out_hbm.at[pl.ds(out0 + r, CHUNK)])
            return carry
        lax.fori_loop(0, RPT // CHUNK, wloop, 0)
        if with_deg:
            pltpu.sync_copy(deg_sh.at[pl.ds(row0, RPT)], dwb)
            pltpu.sync_copy(dwb, deg_hbm.at[pl.ds(out0, RPT)])

    out_type = [jax.ShapeDtypeStruct((NC * NPAD, D), jnp.float32)]
    scratch = [
        pltpu.VMEM((CPT, CHUNK), jnp.int32),
        pltpu.VMEM((CPT, CHUNK), jnp.int32),
        pltpu.VMEM((CHUNK, D), jnp.float32),
        pltpu.VMEM((16, D), jnp.float32),
    ]
    if with_deg:
        out_type.append(jax.ShapeDtypeStruct((NC * NPAD,), jnp.float32))
        scratch += [
            pltpu.VMEM((CHUNK,), jnp.float32),
            pltpu.VMEM((RPT,), jnp.float32),
        ]
    scratch.append(pltpu.VMEM_SHARED((NPAD, D), jnp.float32))
    if with_deg:
        scratch.append(pltpu.VMEM_SHARED((NPAD,), jnp.float32))
    scratch.append(pltpu.SemaphoreType.DMA)
    return pl.kernel(body, mesh=_MESH, out_type=out_type,
                     scratch_types=scratch)


_segsum_deg = _make_segsum(True)
_segsum = _make_segsum(False)

_BLK = 1000
_GRID = N // _BLK


def _mm_body(x_ref, w_ref, o_ref):
    o_ref[...] = jnp.dot(x_ref[...], w_ref[...],
                         preferred_element_type=jnp.float32)


def _tc_lead(x, wa):
    return pl.pallas_call(
        _mm_body,
        grid=(_GRID,),
        in_specs=[pl.BlockSpec((_BLK, D), lambda i: (i, 0)),
                  pl.BlockSpec((D, 3 * D), lambda i: (0, 0))],
        out_specs=pl.BlockSpec((_BLK, 3 * D), lambda i: (i, 0)),
        out_shape=jax.ShapeDtypeStruct((N, 3 * D), jnp.float32),
    )(x, wa)


def _prelu(v, a):
    return jnp.where(v >= 0, v, a * v)


def _mid_body(agg_ref, deg_ref, r0_ref, xs_ref, b0_ref, a0_ref, wc_ref,
              y1_ref, r1_ref):
    deg = deg_ref[0] + deg_ref[1]
    invd = 1.0 / jnp.maximum(deg, 1.0)
    h0 = (agg_ref[0] + agg_ref[1]) * invd + b0_ref[...] + r0_ref[...]
    a0 = a0_ref[...]
    z = _prelu(_prelu(h0, a0), a0) + xs_ref[...]
    y = jnp.dot(z, wc_ref[...], preferred_element_type=jnp.float32)
    y1_ref[...] = y[:, :D]
    r1_ref[...] = y[:, D:]


def _tc_mid(agg, deg, r0, xs, b0, a0, wc):
    return pl.pallas_call(
        _mid_body,
        grid=(_GRID,),
        in_specs=[pl.BlockSpec((NC, _BLK, D), lambda i: (0, i, 0)),
                  pl.BlockSpec((NC, _BLK, 1), lambda i: (0, i, 0)),
                  pl.BlockSpec((_BLK, D), lambda i: (i, 0)),
                  pl.BlockSpec((_BLK, D), lambda i: (i, 0)),
                  pl.BlockSpec((1, D), lambda i: (0, 0)),
                  pl.BlockSpec((1, D), lambda i: (0, 0)),
                  pl.BlockSpec((D, 2 * D), lambda i: (0, 0))],
        out_specs=[pl.BlockSpec((_BLK, D), lambda i: (i, 0)),
                   pl.BlockSpec((_BLK, D), lambda i: (i, 0))],
        out_shape=[jax.ShapeDtypeStruct((N, D), jnp.float32),
                   jax.ShapeDtypeStruct((N, D), jnp.float32)],
    )(agg, deg, r0, xs, b0, a0, wc)


def _fin_body(agg_ref, deg_ref, r1_ref, b1_ref, a1_ref, o_ref):
    deg = deg_ref[0] + deg_ref[1]
    invd = 1.0 / jnp.maximum(deg, 1.0)
    h = (agg_ref[0] + agg_ref[1]) * invd + b1_ref[...] + r1_ref[...]
    o_ref[...] = _prelu(h, a1_ref[...])


def _tc_fin(agg, deg, r1, b1, a1):
    return pl.pallas_call(
        _fin_body,
        grid=(_GRID,),
        in_specs=[pl.BlockSpec((NC, _BLK, D), lambda i: (0, i, 0)),
                  pl.BlockSpec((NC, _BLK, 1), lambda i: (0, i, 0)),
                  pl.BlockSpec((_BLK, D), lambda i: (i, 0)),
                  pl.BlockSpec((1, D), lambda i: (0, 0)),
                  pl.BlockSpec((1, D), lambda i: (0, 0))],
        out_specs=pl.BlockSpec((_BLK, D), lambda i: (i, 0)),
        out_shape=jax.ShapeDtypeStruct((N, D), jnp.float32),
    )(agg, deg, r1, b1, a1)


def kernel(x, edge_index, edge_weight, W0l, b0l, W0r, W1l, b1l, W1r,
           Wskip, a0, a1):
    del edge_weight
    src = edge_index[0].astype(jnp.int32)
    dst = edge_index[1].astype(jnp.int32)
    packed = jax.lax.shift_left(src, 14) | dst
    pad = EPAD - E
    edges_p = jnp.concatenate(
        [packed, jnp.full((pad,), N, jnp.int32)]).reshape(NW, CPT, CHUNK)

    wa = jnp.concatenate([W0l, W0r, Wskip], axis=1)
    ya = _tc_lead(x, wa)
    y0 = ya[:, :D]
    r0 = ya[:, D:2 * D]
    xs = ya[:, 2 * D:]

    agg0, deg = _segsum_deg(y0, edges_p)
    agg0 = agg0.reshape(NC, NPAD, D)
    degr = deg.reshape(NC, NPAD, 1)

    wc = jnp.concatenate([W1l, W1r], axis=1)
    y1, r1 = _tc_mid(agg0, degr, r0, xs, b0l.reshape(1, D),
                     a0.reshape(1, D), wc)

    (agg1,) = _segsum(y1, edges_p)
    agg1 = agg1.reshape(NC, NPAD, D)

    return _tc_fin(agg1, degr, r1, b1l.reshape(1, D), a1.reshape(1, D))


# speedup vs baseline: 1.2220x; 1.2220x over previous
import jax
import jax.numpy as jnp
from jax import lax
from jax.experimental import pallas as pl
from jax.experimental.pallas import tpu as pltpu
from jax.experimental.pallas import tpu_sc as plsc

N = 10000
D = 128
E = 320000
NPAD = 10240
NC = 2
NS = 16
NW = NC * NS
CHUNK = 128
CPT = 80
EPAD = NW * CPT * CHUNK
RPT = NPAD // NS

_MESH = plsc.VectorSubcoreMesh(core_axis_name="c", subcore_axis_name="s")


def _make_segsum(with_deg):
    def body(y_hbm, edges_hbm, out_hbm, *rest):
        if with_deg:
            (deg_hbm, src_v, dst_v, rows_v, zb, ones_v, dwb, acc_sh,
             deg_sh, sem_a, sem_d) = rest
        else:
            (src_v, dst_v, rows_v, zb, acc_sh, sem_a) = rest
        c = lax.axis_index("c")
        s = lax.axis_index("s")
        wid = c * NS + s
        row0 = s * RPT

        for i in range(16):
            for j in range(D // 16):
                zb[i, pl.ds(j * 16, 16)] = jnp.zeros((16,), jnp.float32)
        if with_deg:
            for j in range(CHUNK // 16):
                ones_v[pl.ds(j * 16, 16)] = jnp.ones((16,), jnp.float32)
            for j in range(RPT // 16):
                dwb[pl.ds(j * 16, 16)] = jnp.zeros((16,), jnp.float32)

        def zloop(i, carry):
            pltpu.sync_copy(zb, acc_sh.at[pl.ds(row0 + i * 16, 16)])
            return carry
        lax.fori_loop(0, RPT // 16, zloop, 0)
        if with_deg:
            pltpu.sync_copy(dwb, deg_sh.at[pl.ds(row0, RPT)])

        pltpu.sync_copy(edges_hbm.at[wid], dst_v)

        def uloop(i, carry):
            def inner(j, carry2):
                e = dst_v[i, pl.ds(j * 16, 16)]
                src_v[i, pl.ds(j * 16, 16)] = jax.lax.shift_right_logical(
                    e, jnp.full((16,), 14, jnp.int32))
                dst_v[i, pl.ds(j * 16, 16)] = jax.lax.bitwise_and(
                    e, jnp.full((16,), 16383, jnp.int32))
                return carry2
            return lax.fori_loop(0, CHUNK // 16, inner, carry)
        lax.fori_loop(0, CPT, uloop, 0)
        plsc.subcore_barrier()

        def eloop(j, carry):
            pltpu.async_copy(y_hbm.at[src_v.at[j]], rows_v, sem_a).wait()
            if with_deg:
                pltpu.async_copy(ones_v, deg_sh.at[dst_v.at[j]], sem_d,
                                 add=True)
            pltpu.sync_copy(rows_v, acc_sh.at[dst_v.at[j]], add=True)
            return carry
        lax.fori_loop(0, CPT, eloop, 0)
        if with_deg:
            def dloop(j, carry):
                pltpu.make_async_copy(ones_v, deg_sh.at[dst_v.at[0]],
                                      sem_d).wait()
                return carry
            lax.fori_loop(0, CPT, dloop, 0)

        plsc.subcore_barrier()

        out0 = c * NPAD + row0

        def wloop(i, carry):
            r = i * CHUNK
            pltpu.sync_copy(acc_sh.at[pl.ds(row0 + r, CHUNK)], rows_v)
            pltpu.sync_copy(rows_v, out_hbm.at[pl.ds(out0 + r, CHUNK)])
            return carry
        lax.fori_loop(0, RPT // CHUNK, wloop, 0)
        if with_deg:
            pltpu.sync_copy(deg_sh.at[pl.ds(row0, RPT)], dwb)
            pltpu.sync_copy(dwb, deg_hbm.at[pl.ds(out0, RPT)])

    out_type = [jax.ShapeDtypeStruct((NC * NPAD, D), jnp.float32)]
    scratch = [
        pltpu.VMEM((CPT, CHUNK), jnp.int32),
        pltpu.VMEM((CPT, CHUNK), jnp.int32),
        pltpu.VMEM((CHUNK, D), jnp.float32),
        pltpu.VMEM((16, D), jnp.float32),
    ]
    if with_deg:
        out_type.append(jax.ShapeDtypeStruct((NC * NPAD,), jnp.float32))
        scratch += [
            pltpu.VMEM((CHUNK,), jnp.float32),
            pltpu.VMEM((RPT,), jnp.float32),
        ]
    scratch.append(pltpu.VMEM_SHARED((NPAD, D), jnp.float32))
    if with_deg:
        scratch.append(pltpu.VMEM_SHARED((NPAD,), jnp.float32))
    scratch.append(pltpu.SemaphoreType.DMA)
    if with_deg:
        scratch.append(pltpu.SemaphoreType.DMA)
    return pl.kernel(body, mesh=_MESH, out_type=out_type,
                     scratch_types=scratch)


_segsum_deg = _make_segsum(True)
_segsum = _segsum_deg

_BLK = 1000
_GRID = N // _BLK


def _mm_body(x_ref, w_ref, o_ref):
    o_ref[...] = jnp.dot(x_ref[...], w_ref[...],
                         preferred_element_type=jnp.float32)


def _tc_lead(x, wa):
    return pl.pallas_call(
        _mm_body,
        grid=(_GRID,),
        in_specs=[pl.BlockSpec((_BLK, D), lambda i: (i, 0)),
                  pl.BlockSpec((D, 3 * D), lambda i: (0, 0))],
        out_specs=pl.BlockSpec((_BLK, 3 * D), lambda i: (i, 0)),
        out_shape=jax.ShapeDtypeStruct((N, 3 * D), jnp.float32),
    )(x, wa)


def _prelu(v, a):
    return jnp.where(v >= 0, v, a * v)


def _mid_body(agg_ref, deg_ref, r0_ref, xs_ref, b0_ref, a0_ref, wc_ref,
              y1_ref, r1_ref):
    deg = deg_ref[0] + deg_ref[1]
    invd = 1.0 / jnp.maximum(deg, 1.0)
    h0 = (agg_ref[0] + agg_ref[1]) * invd + b0_ref[...] + r0_ref[...]
    a0 = a0_ref[...]
    z = _prelu(_prelu(h0, a0), a0) + xs_ref[...]
    y = jnp.dot(z, wc_ref[...], preferred_element_type=jnp.float32)
    y1_ref[...] = y[:, :D]
    r1_ref[...] = y[:, D:]


def _tc_mid(agg, deg, r0, xs, b0, a0, wc):
    return pl.pallas_call(
        _mid_body,
        grid=(_GRID,),
        in_specs=[pl.BlockSpec((NC, _BLK, D), lambda i: (0, i, 0)),
                  pl.BlockSpec((NC, _BLK, 1), lambda i: (0, i, 0)),
                  pl.BlockSpec((_BLK, D), lambda i: (i, 0)),
                  pl.BlockSpec((_BLK, D), lambda i: (i, 0)),
                  pl.BlockSpec((1, D), lambda i: (0, 0)),
                  pl.BlockSpec((1, D), lambda i: (0, 0)),
                  pl.BlockSpec((D, 2 * D), lambda i: (0, 0))],
        out_specs=[pl.BlockSpec((_BLK, D), lambda i: (i, 0)),
                   pl.BlockSpec((_BLK, D), lambda i: (i, 0))],
        out_shape=[jax.ShapeDtypeStruct((N, D), jnp.float32),
                   jax.ShapeDtypeStruct((N, D), jnp.float32)],
    )(agg, deg, r0, xs, b0, a0, wc)


def _fin_body(agg_ref, deg_ref, r1_ref, b1_ref, a1_ref, o_ref):
    deg = deg_ref[0] + deg_ref[1]
    invd = 1.0 / jnp.maximum(deg, 1.0)
    h = (agg_ref[0] + agg_ref[1]) * invd + b1_ref[...] + r1_ref[...]
    o_ref[...] = _prelu(h, a1_ref[...])


def _tc_fin(agg, deg, r1, b1, a1):
    return pl.pallas_call(
        _fin_body,
        grid=(_GRID,),
        in_specs=[pl.BlockSpec((NC, _BLK, D), lambda i: (0, i, 0)),
                  pl.BlockSpec((NC, _BLK, 1), lambda i: (0, i, 0)),
                  pl.BlockSpec((_BLK, D), lambda i: (i, 0)),
                  pl.BlockSpec((1, D), lambda i: (0, 0)),
                  pl.BlockSpec((1, D), lambda i: (0, 0))],
        out_specs=pl.BlockSpec((_BLK, D), lambda i: (i, 0)),
        out_shape=jax.ShapeDtypeStruct((N, D), jnp.float32),
    )(agg, deg, r1, b1, a1)


def kernel(x, edge_index, edge_weight, W0l, b0l, W0r, W1l, b1l, W1r,
           Wskip, a0, a1):
    del edge_weight
    src = edge_index[0].astype(jnp.int32)
    dst = edge_index[1].astype(jnp.int32)
    packed = jax.lax.shift_left(src, 14) | dst
    pad = EPAD - E
    edges_p = jnp.concatenate(
        [packed, jnp.full((pad,), N, jnp.int32)]).reshape(NW, CPT, CHUNK)

    wa = jnp.concatenate([W0l, W0r, Wskip], axis=1)
    ya = _tc_lead(x, wa)
    y0 = ya[:, :D]
    r0 = ya[:, D:2 * D]
    xs = ya[:, 2 * D:]

    agg0, deg = _segsum_deg(y0, edges_p)
    agg0 = agg0.reshape(NC, NPAD, D)
    degr = deg.reshape(NC, NPAD, 1)

    wc = jnp.concatenate([W1l, W1r], axis=1)
    y1, r1 = _tc_mid(agg0, degr, r0, xs, b0l.reshape(1, D),
                     a0.reshape(1, D), wc)

    agg1, _ = _segsum(y1, edges_p)
    agg1 = agg1.reshape(NC, NPAD, D)

    return _tc_fin(agg1, degr, r1, b1l.reshape(1, D), a1.reshape(1, D))


# speedup vs baseline: 1.2261x; 1.0033x over previous
import jax
import jax.numpy as jnp
from jax import lax
from jax.experimental import pallas as pl
from jax.experimental.pallas import tpu as pltpu
from jax.experimental.pallas import tpu_sc as plsc

N = 10000
D = 128
E = 320000
NPAD = 10240
NC = 2
NS = 16
NW = NC * NS
CHUNK = 128
CPT = 80
EPAD = NW * CPT * CHUNK
RPT = NPAD // NS

_MESH = plsc.VectorSubcoreMesh(core_axis_name="c", subcore_axis_name="s")


def _make_segsum(with_deg):
    def body(y_hbm, edges_hbm, out_hbm, *rest):
        if with_deg:
            (deg_hbm, src_v, dst_v, rows_v, ones_v, dwb, acc_sh,
             deg_sh, sem_a, sem_d) = rest
        else:
            (src_v, dst_v, rows_v, acc_sh, sem_a) = rest
        c = lax.axis_index("c")
        s = lax.axis_index("s")
        wid = c * NS + s
        row0 = s * RPT

        def zfill(i, carry):
            for j in range(D // 16):
                rows_v[i, pl.ds(j * 16, 16)] = jnp.zeros((16,), jnp.float32)
            return carry
        lax.fori_loop(0, CHUNK, zfill, 0)
        if with_deg:
            for j in range(CHUNK // 16):
                ones_v[pl.ds(j * 16, 16)] = jnp.ones((16,), jnp.float32)
            for j in range(RPT // 16):
                dwb[pl.ds(j * 16, 16)] = jnp.zeros((16,), jnp.float32)

        def zloop(i, carry):
            pltpu.sync_copy(rows_v, acc_sh.at[pl.ds(row0 + i * CHUNK, CHUNK)])
            return carry
        lax.fori_loop(0, RPT // CHUNK, zloop, 0)
        if with_deg:
            pltpu.sync_copy(dwb, deg_sh.at[pl.ds(row0, RPT)])

        pltpu.sync_copy(edges_hbm.at[wid], dst_v)

        def uloop(i, carry):
            def inner(j, carry2):
                e = dst_v[i, pl.ds(j * 16, 16)]
                src_v[i, pl.ds(j * 16, 16)] = jax.lax.shift_right_logical(
                    e, jnp.full((16,), 14, jnp.int32))
                dst_v[i, pl.ds(j * 16, 16)] = jax.lax.bitwise_and(
                    e, jnp.full((16,), 16383, jnp.int32))
                return carry2
            return lax.fori_loop(0, CHUNK // 16, inner, carry)
        lax.fori_loop(0, CPT, uloop, 0)
        plsc.subcore_barrier()

        def eloop(j, carry):
            pltpu.async_copy(y_hbm.at[src_v.at[j]], rows_v, sem_a).wait()
            if with_deg:
                pltpu.async_copy(ones_v, deg_sh.at[dst_v.at[j]], sem_d,
                                 add=True)
            pltpu.sync_copy(rows_v, acc_sh.at[dst_v.at[j]], add=True)
            return carry
        lax.fori_loop(0, CPT, eloop, 0)
        if with_deg:
            def dloop(j, carry):
                pltpu.make_async_copy(ones_v, deg_sh.at[dst_v.at[0]],
                                      sem_d).wait()
                return carry
            lax.fori_loop(0, CPT, dloop, 0)

        plsc.subcore_barrier()

        out0 = c * NPAD + row0

        def wloop(i, carry):
            r = i * CHUNK
            pltpu.sync_copy(acc_sh.at[pl.ds(row0 + r, CHUNK)], rows_v)
            pltpu.sync_copy(rows_v, out_hbm.at[pl.ds(out0 + r, CHUNK)])
            return carry
        lax.fori_loop(0, RPT // CHUNK, wloop, 0)
        if with_deg:
            pltpu.sync_copy(deg_sh.at[pl.ds(row0, RPT)], dwb)
            pltpu.sync_copy(dwb, deg_hbm.at[pl.ds(out0, RPT)])

    out_type = [jax.ShapeDtypeStruct((NC * NPAD, D), jnp.float32)]
    scratch = [
        pltpu.VMEM((CPT, CHUNK), jnp.int32),
        pltpu.VMEM((CPT, CHUNK), jnp.int32),
        pltpu.VMEM((CHUNK, D), jnp.float32),
    ]
    if with_deg:
        out_type.append(jax.ShapeDtypeStruct((NC * NPAD,), jnp.float32))
        scratch += [
            pltpu.VMEM((CHUNK,), jnp.float32),
            pltpu.VMEM((RPT,), jnp.float32),
        ]
    scratch.append(pltpu.VMEM_SHARED((NPAD, D), jnp.float32))
    if with_deg:
        scratch.append(pltpu.VMEM_SHARED((NPAD,), jnp.float32))
    scratch.append(pltpu.SemaphoreType.DMA)
    if with_deg:
        scratch.append(pltpu.SemaphoreType.DMA)
    return pl.kernel(body, mesh=_MESH, out_type=out_type,
                     scratch_types=scratch)


_segsum_deg = _make_segsum(True)
_segsum = _segsum_deg

_BLK = 1000
_GRID = N // _BLK


def _mm_body(x_ref, w_ref, o_ref):
    o_ref[...] = jnp.dot(x_ref[...], w_ref[...],
                         preferred_element_type=jnp.float32)


def _tc_lead(x, wa):
    return pl.pallas_call(
        _mm_body,
        grid=(_GRID,),
        in_specs=[pl.BlockSpec((_BLK, D), lambda i: (i, 0)),
                  pl.BlockSpec((D, 3 * D), lambda i: (0, 0))],
        out_specs=pl.BlockSpec((_BLK, 3 * D), lambda i: (i, 0)),
        out_shape=jax.ShapeDtypeStruct((N, 3 * D), jnp.float32),
    )(x, wa)


def _prelu(v, a):
    return jnp.where(v >= 0, v, a * v)


def _mid_body(agg_ref, deg_ref, r0_ref, xs_ref, b0_ref, a0_ref, wc_ref,
              y1_ref, r1_ref):
    deg = deg_ref[0] + deg_ref[1]
    invd = 1.0 / jnp.maximum(deg, 1.0)
    h0 = (agg_ref[0] + agg_ref[1]) * invd + b0_ref[...] + r0_ref[...]
    a0 = a0_ref[...]
    z = _prelu(_prelu(h0, a0), a0) + xs_ref[...]
    y = jnp.dot(z, wc_ref[...], preferred_element_type=jnp.float32)
    y1_ref[...] = y[:, :D]
    r1_ref[...] = y[:, D:]


def _tc_mid(agg, deg, r0, xs, b0, a0, wc):
    return pl.pallas_call(
        _mid_body,
        grid=(_GRID,),
        in_specs=[pl.BlockSpec((NC, _BLK, D), lambda i: (0, i, 0)),
                  pl.BlockSpec((NC, _BLK, 1), lambda i: (0, i, 0)),
                  pl.BlockSpec((_BLK, D), lambda i: (i, 0)),
                  pl.BlockSpec((_BLK, D), lambda i: (i, 0)),
                  pl.BlockSpec((1, D), lambda i: (0, 0)),
                  pl.BlockSpec((1, D), lambda i: (0, 0)),
                  pl.BlockSpec((D, 2 * D), lambda i: (0, 0))],
        out_specs=[pl.BlockSpec((_BLK, D), lambda i: (i, 0)),
                   pl.BlockSpec((_BLK, D), lambda i: (i, 0))],
        out_shape=[jax.ShapeDtypeStruct((N, D), jnp.float32),
                   jax.ShapeDtypeStruct((N, D), jnp.float32)],
    )(agg, deg, r0, xs, b0, a0, wc)


def _fin_body(agg_ref, deg_ref, r1_ref, b1_ref, a1_ref, o_ref):
    deg = deg_ref[0] + deg_ref[1]
    invd = 1.0 / jnp.maximum(deg, 1.0)
    h = (agg_ref[0] + agg_ref[1]) * invd + b1_ref[...] + r1_ref[...]
    o_ref[...] = _prelu(h, a1_ref[...])


def _tc_fin(agg, deg, r1, b1, a1):
    return pl.pallas_call(
        _fin_body,
        grid=(_GRID,),
        in_specs=[pl.BlockSpec((NC, _BLK, D), lambda i: (0, i, 0)),
                  pl.BlockSpec((NC, _BLK, 1), lambda i: (0, i, 0)),
                  pl.BlockSpec((_BLK, D), lambda i: (i, 0)),
                  pl.BlockSpec((1, D), lambda i: (0, 0)),
                  pl.BlockSpec((1, D), lambda i: (0, 0))],
        out_specs=pl.BlockSpec((_BLK, D), lambda i: (i, 0)),
        out_shape=jax.ShapeDtypeStruct((N, D), jnp.float32),
    )(agg, deg, r1, b1, a1)


def kernel(x, edge_index, edge_weight, W0l, b0l, W0r, W1l, b1l, W1r,
           Wskip, a0, a1):
    del edge_weight
    src = edge_index[0].astype(jnp.int32)
    dst = edge_index[1].astype(jnp.int32)
    packed = jax.lax.shift_left(src, 14) | dst
    pad = EPAD - E
    edges_p = jnp.concatenate(
        [packed, jnp.full((pad,), N, jnp.int32)]).reshape(NW, CPT, CHUNK)

    wa = jnp.concatenate([W0l, W0r, Wskip], axis=1)
    ya = _tc_lead(x, wa)
    y0 = ya[:, :D]
    r0 = ya[:, D:2 * D]
    xs = ya[:, 2 * D:]

    agg0, deg = _segsum_deg(y0, edges_p)
    agg0 = agg0.reshape(NC, NPAD, D)
    degr = deg.reshape(NC, NPAD, 1)

    wc = jnp.concatenate([W1l, W1r], axis=1)
    y1, r1 = _tc_mid(agg0, degr, r0, xs, b0l.reshape(1, D),
                     a0.reshape(1, D), wc)

    agg1, _ = _segsum(y1, edges_p)
    agg1 = agg1.reshape(NC, NPAD, D)

    return _tc_fin(agg1, degr, r1, b1l.reshape(1, D), a1.reshape(1, D))
